# Initial kernel scaffold; baseline (speedup 1.0000x reference)
import jax
import jax.numpy as jnp
from jax.experimental import pallas as pl


def kernel(node_feature, node_embeddings):
    raise NotImplementedError("write your pallas kernel here")


# speedup vs baseline: 7.5983x; 7.5983x over previous
import functools

import jax
import jax.numpy as jnp
from jax import lax
from jax.experimental import pallas as pl
from jax.experimental.pallas import tpu as pltpu
from jax.experimental.pallas import tpu_sc as plsc

TOPK_K = 32


def _topk_body(emb_ref, feat_ref, idx_ref, gidx_ref):
    b = pl.program_id(0)
    emb = emb_ref[...]
    feat = feat_ref[0]
    n = feat.shape[0]
    s = lax.dot_general(
        emb, feat, (((1,), (1,)), ((), ())),
        preferred_element_type=jnp.float32,
        precision=lax.Precision.DEFAULT,
    )
    m_dim = s.shape[0]
    lane_iota = lax.broadcasted_iota(jnp.int32, (m_dim, n), 1)
    neg_inf = jnp.float32(jnp.finfo(jnp.float32).min)
    cols = []
    for _ in range(TOPK_K):
        m = jnp.max(s, axis=1, keepdims=True)
        cand = jnp.where(s == m, lane_iota, jnp.int32(n))
        a = jnp.min(cand, axis=1, keepdims=True)
        cols.append(a)
        s = jnp.where(lane_iota == a, neg_inf, s)
    idx = jnp.concatenate(cols, axis=1)
    idx_ref[0] = idx
    gidx_ref[0] = idx + b * n


def _topk_call(node_feature, node_embeddings):
    B, N, D = node_feature.shape
    M = node_embeddings.shape[0]
    out_shapes = (
        jax.ShapeDtypeStruct((B, M, TOPK_K), jnp.int32),
        jax.ShapeDtypeStruct((B, M, TOPK_K), jnp.int32),
    )
    return pl.pallas_call(
        _topk_body,
        grid=(B,),
        in_specs=[
            pl.BlockSpec((M, D), lambda b: (0, 0)),
            pl.BlockSpec((1, N, D), lambda b: (b, 0, 0)),
        ],
        out_specs=(
            pl.BlockSpec((1, M, TOPK_K), lambda b: (b, 0, 0)),
            pl.BlockSpec((1, M, TOPK_K), lambda b: (b, 0, 0)),
        ),
        out_shape=out_shapes,
        compiler_params=pltpu.CompilerParams(
            dimension_semantics=("arbitrary",),
        ),
    )(node_embeddings, node_feature)


def _make_sc_gather(R, V, D):
    info = plsc.get_sparse_core_info()
    NC, NS = info.num_cores, info.num_subcores
    NW = NC * NS
    rows_per_w = R // NW
    CH = 128
    n_ch = rows_per_w // CH
    mesh = plsc.VectorSubcoreMesh(core_axis_name="c", subcore_axis_name="s")

    @functools.partial(
        pl.kernel,
        mesh=mesh,
        out_type=jax.ShapeDtypeStruct((R, D), jnp.float32),
        scratch_types=[
            pltpu.VMEM((CH,), jnp.int32),
            pltpu.VMEM((CH, D), jnp.float32),
            pltpu.SemaphoreType.DMA,
        ],
    )
    def gather_kernel(table_hbm, gidx_hbm, out_hbm, idx_v, rows_v, sem):
        wid = lax.axis_index("s") * NC + lax.axis_index("c")
        base = wid * rows_per_w

        def body(c, carry):
            off = base + c * CH
            pltpu.sync_copy(gidx_hbm.at[pl.ds(off, CH)], idx_v)
            pltpu.async_copy(table_hbm.at[idx_v], rows_v, sem).wait()
            pltpu.sync_copy(rows_v, out_hbm.at[pl.ds(off, CH)])
            return carry

        lax.fori_loop(0, n_ch, body, 0)

    return gather_kernel


def kernel(node_feature, node_embeddings):
    B, N, D = node_feature.shape
    M = node_embeddings.shape[0]
    K = TOPK_K
    idx, gidx = _topk_call(node_feature, node_embeddings)
    table = node_feature.reshape(B * N, D)
    sel = _make_sc_gather(B * M * K, B * N, D)(table, gidx.reshape(-1))
    sel = sel.reshape(B, M, K, D)
    batch_indices = jnp.broadcast_to(
        jnp.arange(B, dtype=idx.dtype)[:, None, None], (B, M, K)
    )
    return sel, batch_indices, idx


# speedup vs baseline: 10.0917x; 1.3282x over previous
import functools

import jax
import jax.numpy as jnp
from jax import lax
from jax.experimental import pallas as pl
from jax.experimental.pallas import tpu as pltpu
from jax.experimental.pallas import tpu_sc as plsc

TOPK_K = 32


def _topk_body(emb_ref, feat_ref, idx_ref, gidx_ref):
    b = pl.program_id(0)
    emb = emb_ref[...]
    feat = feat_ref[0]
    n = feat.shape[0]
    s = lax.dot_general(
        emb, feat, (((1,), (1,)), ((), ())),
        preferred_element_type=jnp.float32,
        precision=lax.Precision.DEFAULT,
    )
    m_dim = s.shape[0]
    lane_iota = lax.broadcasted_iota(jnp.int32, (m_dim, n), 1)
    neg_inf = jnp.float32(jnp.finfo(jnp.float32).min)
    cols = []
    for _ in range(TOPK_K):
        m = jnp.max(s, axis=1, keepdims=True)
        eqm = s == m
        cand = jnp.where(eqm, lane_iota, jnp.int32(n))
        a = jnp.min(cand, axis=1, keepdims=True)
        cols.append(a)
        s = jnp.where(eqm, neg_inf, s)
    idx = jnp.concatenate(cols, axis=1)
    idx_ref[0] = idx
    gidx_ref[0] = idx + b * n


def _topk_call(node_feature, node_embeddings):
    B, N, D = node_feature.shape
    M = node_embeddings.shape[0]
    out_shapes = (
        jax.ShapeDtypeStruct((B, M, TOPK_K), jnp.int32),
        jax.ShapeDtypeStruct((B, M, TOPK_K), jnp.int32),
    )
    return pl.pallas_call(
        _topk_body,
        grid=(B,),
        in_specs=[
            pl.BlockSpec((M, D), lambda b: (0, 0)),
            pl.BlockSpec((1, N, D), lambda b: (b, 0, 0)),
        ],
        out_specs=(
            pl.BlockSpec((1, M, TOPK_K), lambda b: (b, 0, 0)),
            pl.BlockSpec((1, M, TOPK_K), lambda b: (b, 0, 0)),
        ),
        out_shape=out_shapes,
        compiler_params=pltpu.CompilerParams(
            dimension_semantics=("arbitrary",),
        ),
    )(node_embeddings, node_feature)


def _make_sc_gather(R, V, D):
    info = plsc.get_sparse_core_info()
    NC, NS = info.num_cores, info.num_subcores
    NW = NC * NS
    rows_per_w = R // NW
    CH = 128
    n_ch = rows_per_w // CH
    mesh = plsc.VectorSubcoreMesh(core_axis_name="c", subcore_axis_name="s")

    @functools.partial(
        pl.kernel,
        mesh=mesh,
        out_type=jax.ShapeDtypeStruct((R, D), jnp.float32),
        scratch_types=[
            pltpu.VMEM((CH,), jnp.int32),
            pltpu.VMEM((CH, D), jnp.float32),
            pltpu.SemaphoreType.DMA,
        ],
    )
    def gather_kernel(table_hbm, gidx_hbm, out_hbm, idx_v, rows_v, sem):
        wid = lax.axis_index("s") * NC + lax.axis_index("c")
        base = wid * rows_per_w

        def body(c, carry):
            off = base + c * CH
            pltpu.sync_copy(gidx_hbm.at[pl.ds(off, CH)], idx_v)
            pltpu.async_copy(table_hbm.at[idx_v], rows_v, sem).wait()
            pltpu.sync_copy(rows_v, out_hbm.at[pl.ds(off, CH)])
            return carry

        lax.fori_loop(0, n_ch, body, 0)

    return gather_kernel


def kernel(node_feature, node_embeddings):
    B, N, D = node_feature.shape
    M = node_embeddings.shape[0]
    K = TOPK_K
    idx, gidx = _topk_call(node_feature, node_embeddings)
    table = node_feature.reshape(B * N, D)
    sel = _make_sc_gather(B * M * K, B * N, D)(table, gidx.reshape(-1))
    sel = sel.reshape(B, M, K, D)
    batch_indices = jnp.broadcast_to(
        jnp.arange(B, dtype=idx.dtype)[:, None, None], (B, M, K)
    )
    return sel, batch_indices, idx


# speedup vs baseline: 10.6430x; 1.0546x over previous
import functools

import jax
import jax.numpy as jnp
from jax import lax
from jax.experimental import pallas as pl
from jax.experimental.pallas import tpu as pltpu
from jax.experimental.pallas import tpu_sc as plsc

TOPK_K = 32


def _topk_body(emb_ref, feat_ref, idx_ref, gidx_ref):
    b = pl.program_id(0)
    emb = emb_ref[...]
    feat = feat_ref[0]
    n = feat.shape[0]
    s = lax.dot_general(
        emb, feat, (((1,), (1,)), ((), ())),
        preferred_element_type=jnp.float32,
        precision=lax.Precision.DEFAULT,
    )
    m_dim = s.shape[0]
    lane_iota_f = lax.broadcasted_iota(jnp.int32, (m_dim, n), 1).astype(jnp.float32)
    neg_inf = jnp.float32(jnp.finfo(jnp.float32).min)
    big_f = jnp.float32(n)
    cols = []
    for _ in range(TOPK_K):
        m = jnp.max(s, axis=1, keepdims=True)
        eqm = s == m
        cand = jnp.where(eqm, lane_iota_f, big_f)
        a = jnp.min(cand, axis=1, keepdims=True)
        cols.append(a)
        s = jnp.where(eqm, neg_inf, s)
    idx = jnp.concatenate(cols, axis=1).astype(jnp.int32)
    idx_ref[0] = idx
    gidx_ref[0] = idx + b * n


def _topk_call(node_feature, node_embeddings):
    B, N, D = node_feature.shape
    M = node_embeddings.shape[0]
    out_shapes = (
        jax.ShapeDtypeStruct((B, M, TOPK_K), jnp.int32),
        jax.ShapeDtypeStruct((B, M, TOPK_K), jnp.int32),
    )
    return pl.pallas_call(
        _topk_body,
        grid=(B,),
        in_specs=[
            pl.BlockSpec((M, D), lambda b: (0, 0)),
            pl.BlockSpec((1, N, D), lambda b: (b, 0, 0)),
        ],
        out_specs=(
            pl.BlockSpec((1, M, TOPK_K), lambda b: (b, 0, 0)),
            pl.BlockSpec((1, M, TOPK_K), lambda b: (b, 0, 0)),
        ),
        out_shape=out_shapes,
        compiler_params=pltpu.CompilerParams(
            dimension_semantics=("arbitrary",),
        ),
    )(node_embeddings, node_feature)


def _make_sc_gather(R, V, D):
    info = plsc.get_sparse_core_info()
    NC, NS = info.num_cores, info.num_subcores
    NW = NC * NS
    rows_per_w = R // NW
    CH = 128
    n_ch = rows_per_w // CH
    mesh = plsc.VectorSubcoreMesh(core_axis_name="c", subcore_axis_name="s")

    @functools.partial(
        pl.kernel,
        mesh=mesh,
        out_type=jax.ShapeDtypeStruct((R, D), jnp.float32),
        scratch_types=[
            pltpu.VMEM((CH,), jnp.int32),
            pltpu.VMEM((CH, D), jnp.float32),
            pltpu.SemaphoreType.DMA,
        ],
    )
    def gather_kernel(table_hbm, gidx_hbm, out_hbm, idx_v, rows_v, sem):
        wid = lax.axis_index("s") * NC + lax.axis_index("c")
        base = wid * rows_per_w

        def body(c, carry):
            off = base + c * CH
            pltpu.sync_copy(gidx_hbm.at[pl.ds(off, CH)], idx_v)
            pltpu.async_copy(table_hbm.at[idx_v], rows_v, sem).wait()
            pltpu.sync_copy(rows_v, out_hbm.at[pl.ds(off, CH)])
            return carry

        lax.fori_loop(0, n_ch, body, 0)

    return gather_kernel


def kernel(node_feature, node_embeddings):
    B, N, D = node_feature.shape
    M = node_embeddings.shape[0]
    K = TOPK_K
    idx, gidx = _topk_call(node_feature, node_embeddings)
    table = node_feature.reshape(B * N, D)
    sel = _make_sc_gather(B * M * K, B * N, D)(table, gidx.reshape(-1))
    sel = sel.reshape(B, M, K, D)
    batch_indices = jnp.broadcast_to(
        jnp.arange(B, dtype=idx.dtype)[:, None, None], (B, M, K)
    )
    return sel, batch_indices, idx
